# baseline (speedup 1.0000x reference)
import jax
import jax.numpy as jnp
from jax.experimental import pallas as pl

_DEPTH = 1000
_BD = 200


def _onehot_block(idxt_ref, out_ref):
    d0 = pl.program_id(1) * _BD
    idxt = idxt_ref[0]
    b1, b = idxt.shape
    iota = d0 + jax.lax.broadcasted_iota(jnp.int32, (b1, _BD, b), 1)
    out_ref[...] = (idxt[:, None, :] == iota).astype(jnp.float32)


def kernel(inputs):
    x = inputs.astype(jnp.int32)
    if x.ndim == 3:
        x = x[:, :, 0]
    b, s = x.shape
    xt = x.T.reshape(s, 1, b)
    out = pl.pallas_call(
        _onehot_block,
        grid=(s, _DEPTH // _BD),
        in_specs=[pl.BlockSpec((1, 1, b), lambda i, j: (i, 0, 0))],
        out_specs=pl.BlockSpec((1, _BD, b), lambda i, j: (i, j, 0)),
        out_shape=jax.ShapeDtypeStruct((s, _DEPTH, b), jnp.float32),
    )(xt)
    return jnp.transpose(out, (2, 0, 1))


# speedup vs baseline: 2.1162x; 2.1162x over previous
import jax
import jax.numpy as jnp
from jax.experimental import pallas as pl

_DEPTH = 1000
_B1 = 1


def _onehot_block(idxt_ref, out_ref):
    idxt = idxt_ref[0]
    b1, b = idxt.shape
    iota = jax.lax.broadcasted_iota(jnp.int32, (b1, _DEPTH, b), 1)
    out_ref[...] = (idxt[:, None, :] == iota).astype(jnp.float32)


def kernel(inputs):
    x = inputs.astype(jnp.int32)
    if x.ndim == 3:
        x = x[:, :, 0]
    b, s = x.shape
    g = s // _B1
    xt = x.T.reshape(g, _B1, b)
    out = pl.pallas_call(
        _onehot_block,
        grid=(g,),
        in_specs=[pl.BlockSpec((1, _B1, b), lambda i: (i, 0, 0))],
        out_specs=pl.BlockSpec((_B1, _DEPTH, b), lambda i: (i, 0, 0)),
        out_shape=jax.ShapeDtypeStruct((s, _DEPTH, b), jnp.float32),
    )(xt)
    return jnp.transpose(out, (2, 0, 1))
